# baseline (speedup 1.0000x reference)
import functools

import jax
import jax.numpy as jnp
from jax import lax
from jax.experimental import pallas as pl
from jax.experimental.pallas import tpu as pltpu
from jax.experimental.pallas import tpu_sc as plsc

L = 16
NW = 32
B = 128
NC = 8
T = 2000

_PARAMS = pltpu.CompilerParams(
    needs_layout_passes=False, use_tc_tiling_on_sc=False)


def _make_format_call(C, V):
    VW = V // NW
    NB = VW // T
    NJ = T // L

    mesh = plsc.VectorSubcoreMesh(core_axis_name="c", subcore_axis_name="s")

    @functools.partial(
        pl.kernel,
        mesh=mesh,
        compiler_params=_PARAMS,
        out_type=jax.ShapeDtypeStruct((V, L), jnp.float32),
        scratch_types=[
            pltpu.VMEM((C, T), jnp.float32),
            pltpu.VMEM((T, L), jnp.float32),
            pltpu.SemaphoreType.DMA,
        ],
    )
    def fmt(grid_hbm, table_hbm, ch_v, out_v, sem):
        wid = lax.axis_index("s") * 2 + lax.axis_index("c")
        v0 = wid * VW
        iota = lax.iota(jnp.int32, L)
        zero = iota - iota
        zf = jnp.zeros((L,), jnp.float32)

        def zf_body(j, carry):
            out_v[j, :] = zf
            return carry

        lax.fori_loop(0, T, zf_body, 0)

        def block_body(bi, carry):
            vb = v0 + bi * T
            cps = [pltpu.async_copy(
                       grid_hbm.at[pl.ds(c * V + vb, T)], ch_v.at[c], sem)
                   for c in range(C)]
            for cp in cps:
                cp.wait()
            for j in range(NJ):
                vv = iota + j * L
                sl = pl.ds(j * L, L)
                for c in range(C):
                    plsc.store_scatter(out_v, [vv, zero + c], ch_v[c, sl])
            pltpu.sync_copy(out_v, table_hbm.at[pl.ds(vb, T), :])
            return carry

        lax.fori_loop(0, NB, block_body, 0)

    return fmt


def _make_sample_call(N, C, D, H, W):
    PW = N // NW
    NB = PW // B
    NG = B // L

    mesh = plsc.VectorSubcoreMesh(core_axis_name="c", subcore_axis_name="s")

    P = 4
    LEAD = 3

    @functools.partial(
        pl.kernel,
        mesh=mesh,
        compiler_params=_PARAMS,
        out_type=jax.ShapeDtypeStruct((N, C), jnp.float32),
        scratch_types=[
            pltpu.VMEM((6 * L,), jnp.float32),
            pltpu.VMEM((P, B, 3), jnp.float32),
            pltpu.VMEM((P, NC, B), jnp.int32),
            pltpu.VMEM((P, NC, B), jnp.float32),
            pltpu.VMEM((P, NC * B, L), jnp.float32),
            pltpu.VMEM((P, B, C), jnp.float32),
            pltpu.SemaphoreType.DMA((P,)),
            pltpu.SemaphoreType.DMA((P,)),
        ],
    )
    def launch(xyz_hbm, table_hbm, params_hbm, out_hbm,
               par_v, xyz_v, idx_v, w_v, rows_v, out_v, gsem, osem):
        wid = lax.axis_index("s") * 2 + lax.axis_index("c")
        base = wid * PW
        pltpu.sync_copy(params_hbm, par_v)
        iota = lax.iota(jnp.int32, L)
        zero = iota - iota
        mns = [par_v[pl.ds(a * L, L)] for a in range(3)]
        dns = [par_v[pl.ds((3 + a) * L, L)] for a in range(3)]

        def axis_vals(p, a, S):
            n = (p - mns[a]) / dns[a]
            cc = n * 2.0 - 1.0
            u = (cc + 1.0) * 0.5 * float(S - 1)
            u = jnp.clip(u, 0.0, float(S - 1))
            i0 = u.astype(jnp.int32)
            t = u - i0.astype(jnp.float32)
            i1 = jnp.minimum(i0 + 1, S - 1)
            return i0, i1, t

        def fire(b, p):
            pbase = base + b * B
            pltpu.sync_copy(xyz_hbm.at[pl.ds(pbase, B), :], xyz_v.at[p])
            pfv = zero + p
            for g in range(NG):
                off = g * L
                pg = iota + off
                px = plsc.load_gather(xyz_v, [pfv, pg, zero])
                py = plsc.load_gather(xyz_v, [pfv, pg, zero + 1])
                pz = plsc.load_gather(xyz_v, [pfv, pg, zero + 2])
                sl = pl.ds(off, L)
                iz0, iz1, tz = axis_vals(px, 0, D)
                iy0, iy1, ty = axis_vals(py, 1, H)
                ix0, ix1, tx = axis_vals(pz, 2, W)
                a0 = iz0 * (H * W)
                a1 = iz1 * (H * W)
                b0 = iy0 * W
                b1 = iy1 * W
                r00 = a0 + b0
                r01 = a0 + b1
                r10 = a1 + b0
                r11 = a1 + b1
                idx_v[p, 0, sl] = r00 + ix0
                idx_v[p, 1, sl] = r00 + ix1
                idx_v[p, 2, sl] = r01 + ix0
                idx_v[p, 3, sl] = r01 + ix1
                idx_v[p, 4, sl] = r10 + ix0
                idx_v[p, 5, sl] = r10 + ix1
                idx_v[p, 6, sl] = r11 + ix0
                idx_v[p, 7, sl] = r11 + ix1
                cz = 1.0 - tz
                cy = 1.0 - ty
                cx = 1.0 - tx
                w00 = cz * cy
                w01 = cz * ty
                w10 = tz * cy
                w11 = tz * ty
                w_v[p, 0, sl] = w00 * cx
                w_v[p, 1, sl] = w00 * tx
                w_v[p, 2, sl] = w01 * cx
                w_v[p, 3, sl] = w01 * tx
                w_v[p, 4, sl] = w10 * cx
                w_v[p, 5, sl] = w10 * tx
                w_v[p, 6, sl] = w11 * cx
                w_v[p, 7, sl] = w11 * tx
            for k in range(NC):
                pltpu.async_copy(table_hbm.at[idx_v.at[p, k]],
                                 rows_v.at[p, pl.ds(k * B, B), :],
                                 gsem.at[p])

        def gwait(p):
            for k in range(NC):
                pltpu.make_async_copy(table_hbm.at[idx_v.at[p, k]],
                                      rows_v.at[p, pl.ds(k * B, B), :],
                                      gsem.at[p]).wait()

        def owait(b, p):
            pbase = base + b * B
            pltpu.make_async_copy(out_v.at[p],
                                  out_hbm.at[pl.ds(pbase, B), :],
                                  osem.at[p]).wait()

        def finish(b, p):
            pbase = base + b * B
            gwait(p)
            pfv = zero + p
            for g in range(NG):
                off = g * L
                pv = iota + off
                sl = pl.ds(off, L)
                wks = [w_v[p, k, sl] for k in range(NC)]
                for ch in range(C):
                    chv = zero + ch
                    acc = wks[0] * plsc.load_gather(rows_v, [pfv, pv, chv])
                    for k in range(1, NC):
                        rv = pv + (k * B)
                        acc = acc + wks[k] * plsc.load_gather(
                            rows_v, [pfv, rv, chv])
                    plsc.store_scatter(out_v, [pfv, pv, chv], acc)
            pltpu.async_copy(out_v.at[p],
                             out_hbm.at[pl.ds(pbase, B), :],
                             osem.at[p])

        for j in range(LEAD):
            fire(j, j)

        def block_body(t, carry):
            p = lax.bitwise_and(t, P - 1)
            pn = lax.bitwise_and(t + LEAD, P - 1)

            @pl.when(t + LEAD < NB)
            def _():
                fire(t + LEAD, pn)

            @pl.when(t >= P)
            def _():
                owait(t - P, p)

            finish(t, p)
            return carry

        lax.fori_loop(0, NB, block_body, 0)

        for j in range(NB - P, NB):
            owait(j, j % P)

    return launch


def kernel(xyz, grid, xyz_min, xyz_max):
    N = xyz.shape[0]
    C = grid.shape[1]
    D, H, W = grid.shape[2], grid.shape[3], grid.shape[4]
    V = D * H * W
    one = 1.0 + 0.0 * xyz_min[0]
    grid_flat = grid.reshape(C * V) * one
    table = _make_format_call(C, V)(grid_flat)
    den = xyz_max - xyz_min
    params = jnp.concatenate([
        jnp.broadcast_to(xyz_min[:, None], (3, L)).reshape(-1),
        jnp.broadcast_to(den[:, None], (3, L)).reshape(-1),
    ])
    call = _make_sample_call(N, C, D, H, W)
    return call(xyz, table, params)


# speedup vs baseline: 1.0337x; 1.0337x over previous
import functools

import jax
import jax.numpy as jnp
from jax import lax
from jax.experimental import pallas as pl
from jax.experimental.pallas import tpu as pltpu
from jax.experimental.pallas import tpu_sc as plsc

L = 16
NW = 32
B = 128
NC = 8
T = 2000

_PARAMS = pltpu.CompilerParams(
    needs_layout_passes=False, use_tc_tiling_on_sc=False)


def _make_format_call(C, V):
    VW = V // NW
    NB = VW // T
    NJ = T // L

    mesh = plsc.VectorSubcoreMesh(core_axis_name="c", subcore_axis_name="s")

    @functools.partial(
        pl.kernel,
        mesh=mesh,
        compiler_params=_PARAMS,
        out_type=jax.ShapeDtypeStruct((V, L), jnp.float32),
        scratch_types=[
            pltpu.VMEM((2, C, T), jnp.float32),
            pltpu.VMEM((2, T, L), jnp.float32),
            pltpu.SemaphoreType.DMA((2,)),
            pltpu.SemaphoreType.DMA((2,)),
        ],
    )
    def fmt(grid_hbm, table_hbm, ch_v, out_v, csem, osem):
        wid = lax.axis_index("s") * 2 + lax.axis_index("c")
        v0 = wid * VW
        iota = lax.iota(jnp.int32, L)
        zero = iota - iota
        zf = jnp.zeros((L,), jnp.float32)

        def zf_body(j, carry):
            out_v[lax.div(j, T), lax.rem(j, T), :] = zf
            return carry

        lax.fori_loop(0, 2 * T, zf_body, 0)

        def fire(b, p):
            vb = v0 + b * T
            for c in range(C):
                pltpu.async_copy(grid_hbm.at[pl.ds(c * V + vb, T)],
                                 ch_v.at[p, c], csem.at[p])

        def cwait(p):
            for c in range(C):
                pltpu.make_async_copy(grid_hbm.at[pl.ds(0, T)],
                                      ch_v.at[p, c], csem.at[p]).wait()

        def owait(b, p):
            vb = v0 + b * T
            pltpu.make_async_copy(out_v.at[p],
                                  table_hbm.at[pl.ds(vb, T), :],
                                  osem.at[p]).wait()

        fire(0, 0)

        def block_body(t, carry):
            p = lax.bitwise_and(t, 1)
            q = lax.bitwise_and(t + 1, 1)

            @pl.when(t + 1 < NB)
            def _():
                fire(t + 1, q)

            cwait(p)

            @pl.when(t >= 2)
            def _():
                owait(t - 2, p)

            pfv = zero + p
            for j in range(NJ):
                vv = iota + j * L
                sl = pl.ds(j * L, L)
                for c in range(C):
                    plsc.store_scatter(out_v, [pfv, vv, zero + c],
                                       ch_v[p, c, sl])
            vb = v0 + t * T
            pltpu.async_copy(out_v.at[p],
                             table_hbm.at[pl.ds(vb, T), :], osem.at[p])
            return carry

        lax.fori_loop(0, NB, block_body, 0)

        for j in range(NB - 2, NB):
            owait(j, j % 2)

    return fmt


def _make_sample_call(N, C, D, H, W):
    PW = N // NW
    NB = PW // B
    NG = B // L

    mesh = plsc.VectorSubcoreMesh(core_axis_name="c", subcore_axis_name="s")

    P = 4
    LEAD = 3

    @functools.partial(
        pl.kernel,
        mesh=mesh,
        compiler_params=_PARAMS,
        out_type=jax.ShapeDtypeStruct((N, C), jnp.float32),
        scratch_types=[
            pltpu.VMEM((6 * L,), jnp.float32),
            pltpu.VMEM((P, B, 3), jnp.float32),
            pltpu.VMEM((P, NC, B), jnp.int32),
            pltpu.VMEM((P, NC, B), jnp.float32),
            pltpu.VMEM((P, NC * B, L), jnp.float32),
            pltpu.VMEM((P, B, C), jnp.float32),
            pltpu.SemaphoreType.DMA((P,)),
            pltpu.SemaphoreType.DMA((P,)),
        ],
    )
    def launch(xyz_hbm, table_hbm, params_hbm, out_hbm,
               par_v, xyz_v, idx_v, w_v, rows_v, out_v, gsem, osem):
        wid = lax.axis_index("s") * 2 + lax.axis_index("c")
        base = wid * PW
        pltpu.sync_copy(params_hbm, par_v)
        iota = lax.iota(jnp.int32, L)
        zero = iota - iota
        mns = [par_v[pl.ds(a * L, L)] for a in range(3)]
        dns = [par_v[pl.ds((3 + a) * L, L)] for a in range(3)]

        def axis_vals(p, a, S):
            n = (p - mns[a]) / dns[a]
            cc = n * 2.0 - 1.0
            u = (cc + 1.0) * 0.5 * float(S - 1)
            u = jnp.clip(u, 0.0, float(S - 1))
            i0 = u.astype(jnp.int32)
            t = u - i0.astype(jnp.float32)
            i1 = jnp.minimum(i0 + 1, S - 1)
            return i0, i1, t

        def fire(b, p):
            pbase = base + b * B
            pltpu.sync_copy(xyz_hbm.at[pl.ds(pbase, B), :], xyz_v.at[p])
            pfv = zero + p
            for g in range(NG):
                off = g * L
                pg = iota + off
                px = plsc.load_gather(xyz_v, [pfv, pg, zero])
                py = plsc.load_gather(xyz_v, [pfv, pg, zero + 1])
                pz = plsc.load_gather(xyz_v, [pfv, pg, zero + 2])
                sl = pl.ds(off, L)
                iz0, iz1, tz = axis_vals(px, 0, D)
                iy0, iy1, ty = axis_vals(py, 1, H)
                ix0, ix1, tx = axis_vals(pz, 2, W)
                a0 = iz0 * (H * W)
                a1 = iz1 * (H * W)
                b0 = iy0 * W
                b1 = iy1 * W
                r00 = a0 + b0
                r01 = a0 + b1
                r10 = a1 + b0
                r11 = a1 + b1
                idx_v[p, 0, sl] = r00 + ix0
                idx_v[p, 1, sl] = r00 + ix1
                idx_v[p, 2, sl] = r01 + ix0
                idx_v[p, 3, sl] = r01 + ix1
                idx_v[p, 4, sl] = r10 + ix0
                idx_v[p, 5, sl] = r10 + ix1
                idx_v[p, 6, sl] = r11 + ix0
                idx_v[p, 7, sl] = r11 + ix1
                cz = 1.0 - tz
                cy = 1.0 - ty
                cx = 1.0 - tx
                w00 = cz * cy
                w01 = cz * ty
                w10 = tz * cy
                w11 = tz * ty
                w_v[p, 0, sl] = w00 * cx
                w_v[p, 1, sl] = w00 * tx
                w_v[p, 2, sl] = w01 * cx
                w_v[p, 3, sl] = w01 * tx
                w_v[p, 4, sl] = w10 * cx
                w_v[p, 5, sl] = w10 * tx
                w_v[p, 6, sl] = w11 * cx
                w_v[p, 7, sl] = w11 * tx
            for k in range(NC):
                pltpu.async_copy(table_hbm.at[idx_v.at[p, k]],
                                 rows_v.at[p, pl.ds(k * B, B), :],
                                 gsem.at[p])

        def gwait(p):
            for k in range(NC):
                pltpu.make_async_copy(table_hbm.at[idx_v.at[p, k]],
                                      rows_v.at[p, pl.ds(k * B, B), :],
                                      gsem.at[p]).wait()

        def owait(b, p):
            pbase = base + b * B
            pltpu.make_async_copy(out_v.at[p],
                                  out_hbm.at[pl.ds(pbase, B), :],
                                  osem.at[p]).wait()

        def finish(b, p):
            pbase = base + b * B
            gwait(p)
            pfv = zero + p
            for g in range(NG):
                off = g * L
                pv = iota + off
                sl = pl.ds(off, L)
                wks = [w_v[p, k, sl] for k in range(NC)]
                for ch in range(C):
                    chv = zero + ch
                    acc = wks[0] * plsc.load_gather(rows_v, [pfv, pv, chv])
                    for k in range(1, NC):
                        rv = pv + (k * B)
                        acc = acc + wks[k] * plsc.load_gather(
                            rows_v, [pfv, rv, chv])
                    plsc.store_scatter(out_v, [pfv, pv, chv], acc)
            pltpu.async_copy(out_v.at[p],
                             out_hbm.at[pl.ds(pbase, B), :],
                             osem.at[p])

        for j in range(LEAD):
            fire(j, j)

        def block_body(t, carry):
            p = lax.bitwise_and(t, P - 1)
            pn = lax.bitwise_and(t + LEAD, P - 1)

            @pl.when(t + LEAD < NB)
            def _():
                fire(t + LEAD, pn)

            @pl.when(t >= P)
            def _():
                owait(t - P, p)

            finish(t, p)
            return carry

        lax.fori_loop(0, NB, block_body, 0)

        for j in range(NB - P, NB):
            owait(j, j % P)

    return launch


def kernel(xyz, grid, xyz_min, xyz_max):
    N = xyz.shape[0]
    C = grid.shape[1]
    D, H, W = grid.shape[2], grid.shape[3], grid.shape[4]
    V = D * H * W
    grid_flat = grid.reshape(C * V)
    table = _make_format_call(C, V)(grid_flat)
    den = xyz_max - xyz_min
    params = jnp.concatenate([
        jnp.broadcast_to(xyz_min[:, None], (3, L)).reshape(-1),
        jnp.broadcast_to(den[:, None], (3, L)).reshape(-1),
    ])
    call = _make_sample_call(N, C, D, H, W)
    return call(xyz, table, params)


# speedup vs baseline: 1.0609x; 1.0263x over previous
import functools

import jax
import jax.numpy as jnp
from jax import lax
from jax.experimental import pallas as pl
from jax.experimental.pallas import tpu as pltpu
from jax.experimental.pallas import tpu_sc as plsc

L = 16
NW = 32
B = 128
NC = 8
T = 2000

_PARAMS = pltpu.CompilerParams(
    needs_layout_passes=False, use_tc_tiling_on_sc=False)


def _make_format_call(C, V):
    VW = V // NW
    NB = VW // T
    NJ = T // L

    mesh = plsc.VectorSubcoreMesh(core_axis_name="c", subcore_axis_name="s")

    @functools.partial(
        pl.kernel,
        mesh=mesh,
        compiler_params=_PARAMS,
        out_type=jax.ShapeDtypeStruct((V, L), jnp.float32),
        scratch_types=[
            pltpu.VMEM((2, C, T), jnp.float32),
            pltpu.VMEM((2, T, L), jnp.float32),
            pltpu.SemaphoreType.DMA((2,)),
            pltpu.SemaphoreType.DMA((2,)),
        ],
    )
    def fmt(grid_hbm, table_hbm, ch_v, out_v, csem, osem):
        wid = lax.axis_index("s") * 2 + lax.axis_index("c")
        v0 = wid * VW
        iota = lax.iota(jnp.int32, L)
        zero = iota - iota
        zf = jnp.zeros((L,), jnp.float32)

        def zf_body(j, carry):
            out_v[lax.div(j, T), lax.rem(j, T), :] = zf
            return carry

        lax.fori_loop(0, 2 * T, zf_body, 0)

        def fire(b, p):
            vb = v0 + b * T
            for c in range(C):
                pltpu.async_copy(grid_hbm.at[pl.ds(c * V + vb, T)],
                                 ch_v.at[p, c], csem.at[p])

        def cwait(p):
            for c in range(C):
                pltpu.make_async_copy(grid_hbm.at[pl.ds(0, T)],
                                      ch_v.at[p, c], csem.at[p]).wait()

        def owait(b, p):
            vb = v0 + b * T
            pltpu.make_async_copy(out_v.at[p],
                                  table_hbm.at[pl.ds(vb, T), :],
                                  osem.at[p]).wait()

        fire(0, 0)

        def block_body(t, carry):
            p = lax.bitwise_and(t, 1)
            q = lax.bitwise_and(t + 1, 1)

            @pl.when(t + 1 < NB)
            def _():
                fire(t + 1, q)

            cwait(p)

            @pl.when(t >= 2)
            def _():
                owait(t - 2, p)

            pfv = zero + p
            for j in range(NJ):
                vv = iota + j * L
                sl = pl.ds(j * L, L)
                for c in range(C):
                    plsc.store_scatter(out_v, [pfv, vv, zero + c],
                                       ch_v[p, c, sl])
            vb = v0 + t * T
            pltpu.async_copy(out_v.at[p],
                             table_hbm.at[pl.ds(vb, T), :], osem.at[p])
            return carry

        lax.fori_loop(0, NB, block_body, 0)

        for j in range(NB - 2, NB):
            owait(j, j % 2)

    return fmt


def _make_sample_call(N, C, D, H, W):
    PW = N // NW
    NB = PW // B
    NG = B // L

    mesh = plsc.VectorSubcoreMesh(core_axis_name="c", subcore_axis_name="s")

    P = 4
    LEAD = 3

    @functools.partial(
        pl.kernel,
        mesh=mesh,
        compiler_params=_PARAMS,
        out_type=jax.ShapeDtypeStruct((N, C), jnp.float32),
        scratch_types=[
            pltpu.VMEM((6 * L,), jnp.float32),
            pltpu.VMEM((P, B, 3), jnp.float32),
            pltpu.VMEM((P, NC, B), jnp.int32),
            pltpu.VMEM((P, NC, B), jnp.float32),
            pltpu.VMEM((P, NC * B, L), jnp.float32),
            pltpu.VMEM((P, B, C), jnp.float32),
            pltpu.SemaphoreType.DMA((P,)),
            pltpu.SemaphoreType.DMA((P,)),
            pltpu.SemaphoreType.DMA((P,)),
        ],
    )
    def launch(xyz_hbm, table_hbm, params_hbm, out_hbm,
               par_v, xyz_v, idx_v, w_v, rows_v, out_v, gsem, osem, xsem):
        wid = lax.axis_index("s") * 2 + lax.axis_index("c")
        base = wid * PW
        pltpu.sync_copy(params_hbm, par_v)
        iota = lax.iota(jnp.int32, L)
        zero = iota - iota
        mns = [par_v[pl.ds(a * L, L)] for a in range(3)]
        dns = [par_v[pl.ds((3 + a) * L, L)] for a in range(3)]

        def axis_vals(p, a, S):
            n = (p - mns[a]) / dns[a]
            cc = n * 2.0 - 1.0
            u = (cc + 1.0) * 0.5 * float(S - 1)
            u = jnp.clip(u, 0.0, float(S - 1))
            i0 = u.astype(jnp.int32)
            t = u - i0.astype(jnp.float32)
            i1 = jnp.minimum(i0 + 1, S - 1)
            return i0, i1, t

        def xfire(b, p):
            pbase = base + b * B
            pltpu.async_copy(xyz_hbm.at[pl.ds(pbase, B), :], xyz_v.at[p],
                             xsem.at[p])

        def fire(b, p):
            pbase = base + b * B
            pltpu.make_async_copy(xyz_hbm.at[pl.ds(pbase, B), :],
                                  xyz_v.at[p], xsem.at[p]).wait()
            pfv = zero + p
            for g in range(NG):
                off = g * L
                pg = iota + off
                px = plsc.load_gather(xyz_v, [pfv, pg, zero])
                py = plsc.load_gather(xyz_v, [pfv, pg, zero + 1])
                pz = plsc.load_gather(xyz_v, [pfv, pg, zero + 2])
                sl = pl.ds(off, L)
                iz0, iz1, tz = axis_vals(px, 0, D)
                iy0, iy1, ty = axis_vals(py, 1, H)
                ix0, ix1, tx = axis_vals(pz, 2, W)
                a0 = iz0 * (H * W)
                a1 = iz1 * (H * W)
                b0 = iy0 * W
                b1 = iy1 * W
                r00 = a0 + b0
                r01 = a0 + b1
                r10 = a1 + b0
                r11 = a1 + b1
                idx_v[p, 0, sl] = r00 + ix0
                idx_v[p, 1, sl] = r00 + ix1
                idx_v[p, 2, sl] = r01 + ix0
                idx_v[p, 3, sl] = r01 + ix1
                idx_v[p, 4, sl] = r10 + ix0
                idx_v[p, 5, sl] = r10 + ix1
                idx_v[p, 6, sl] = r11 + ix0
                idx_v[p, 7, sl] = r11 + ix1
                cz = 1.0 - tz
                cy = 1.0 - ty
                cx = 1.0 - tx
                w00 = cz * cy
                w01 = cz * ty
                w10 = tz * cy
                w11 = tz * ty
                w_v[p, 0, sl] = w00 * cx
                w_v[p, 1, sl] = w00 * tx
                w_v[p, 2, sl] = w01 * cx
                w_v[p, 3, sl] = w01 * tx
                w_v[p, 4, sl] = w10 * cx
                w_v[p, 5, sl] = w10 * tx
                w_v[p, 6, sl] = w11 * cx
                w_v[p, 7, sl] = w11 * tx
            for k in range(NC):
                pltpu.async_copy(table_hbm.at[idx_v.at[p, k]],
                                 rows_v.at[p, pl.ds(k * B, B), :],
                                 gsem.at[p])

        def gwait(p):
            pltpu.make_async_copy(table_hbm.at[pl.ds(0, NC * B), :],
                                  rows_v.at[p], gsem.at[p]).wait()

        def owait(b, p):
            pbase = base + b * B
            pltpu.make_async_copy(out_v.at[p],
                                  out_hbm.at[pl.ds(pbase, B), :],
                                  osem.at[p]).wait()

        def finish(b, p):
            pbase = base + b * B
            gwait(p)
            pfv = zero + p
            for g in range(NG):
                off = g * L
                pv = iota + off
                sl = pl.ds(off, L)
                wks = [w_v[p, k, sl] for k in range(NC)]
                for ch in range(C):
                    chv = zero + ch
                    acc = wks[0] * plsc.load_gather(rows_v, [pfv, pv, chv])
                    for k in range(1, NC):
                        rv = pv + (k * B)
                        acc = acc + wks[k] * plsc.load_gather(
                            rows_v, [pfv, rv, chv])
                    plsc.store_scatter(out_v, [pfv, pv, chv], acc)
            pltpu.async_copy(out_v.at[p],
                             out_hbm.at[pl.ds(pbase, B), :],
                             osem.at[p])

        for j in range(P):
            xfire(j, j)
        for j in range(LEAD):
            fire(j, j)

        def block_body(t, carry):
            p = lax.bitwise_and(t, P - 1)
            pn = lax.bitwise_and(t + LEAD, P - 1)

            @pl.when(t + P < NB)
            def _():
                xfire(t + P, p)

            @pl.when(t + LEAD < NB)
            def _():
                fire(t + LEAD, pn)

            @pl.when(t >= P)
            def _():
                owait(t - P, p)

            finish(t, p)
            return carry

        lax.fori_loop(0, NB, block_body, 0)

        for j in range(NB - P, NB):
            owait(j, j % P)

    return launch


def kernel(xyz, grid, xyz_min, xyz_max):
    N = xyz.shape[0]
    C = grid.shape[1]
    D, H, W = grid.shape[2], grid.shape[3], grid.shape[4]
    V = D * H * W
    grid_flat = grid.reshape(C * V)
    table = _make_format_call(C, V)(grid_flat)
    den = xyz_max - xyz_min
    params = jnp.concatenate([
        jnp.broadcast_to(xyz_min[:, None], (3, L)).reshape(-1),
        jnp.broadcast_to(den[:, None], (3, L)).reshape(-1),
    ])
    call = _make_sample_call(N, C, D, H, W)
    return call(xyz, table, params)


# speedup vs baseline: 1.0657x; 1.0045x over previous
import functools

import jax
import jax.numpy as jnp
from jax import lax
from jax.experimental import pallas as pl
from jax.experimental.pallas import tpu as pltpu
from jax.experimental.pallas import tpu_sc as plsc

L = 16
NW = 32
B = 128
NC = 8
T = 2000

_PARAMS = pltpu.CompilerParams(
    needs_layout_passes=False, use_tc_tiling_on_sc=False)


def _make_format_call(C, V):
    VW = V // NW
    NB = VW // T
    NJ = T // L

    mesh = plsc.VectorSubcoreMesh(core_axis_name="c", subcore_axis_name="s")

    @functools.partial(
        pl.kernel,
        mesh=mesh,
        compiler_params=_PARAMS,
        out_type=jax.ShapeDtypeStruct((V, L), jnp.float32),
        scratch_types=[
            pltpu.VMEM((2, C, T), jnp.float32),
            pltpu.VMEM((2, T, L), jnp.float32),
            pltpu.SemaphoreType.DMA((2,)),
            pltpu.SemaphoreType.DMA((2,)),
        ],
    )
    def fmt(grid_hbm, table_hbm, ch_v, out_v, csem, osem):
        wid = lax.axis_index("s") * 2 + lax.axis_index("c")
        v0 = wid * VW
        iota = lax.iota(jnp.int32, L)
        zero = iota - iota
        zf = jnp.zeros((L,), jnp.float32)

        def zf_body(j, carry):
            out_v[lax.div(j, T), lax.rem(j, T), :] = zf
            return carry

        lax.fori_loop(0, 2 * T, zf_body, 0)

        def fire(b, p):
            vb = v0 + b * T
            for c in range(C):
                pltpu.async_copy(grid_hbm.at[pl.ds(c * V + vb, T)],
                                 ch_v.at[p, c], csem.at[p])

        def cwait(p):
            for c in range(C):
                pltpu.make_async_copy(grid_hbm.at[pl.ds(0, T)],
                                      ch_v.at[p, c], csem.at[p]).wait()

        def owait(b, p):
            vb = v0 + b * T
            pltpu.make_async_copy(out_v.at[p],
                                  table_hbm.at[pl.ds(vb, T), :],
                                  osem.at[p]).wait()

        fire(0, 0)

        def block_body(t, carry):
            p = lax.bitwise_and(t, 1)
            q = lax.bitwise_and(t + 1, 1)

            @pl.when(t + 1 < NB)
            def _():
                fire(t + 1, q)

            cwait(p)

            @pl.when(t >= 2)
            def _():
                owait(t - 2, p)

            pfv = zero + p
            for j in range(NJ):
                vv = iota + j * L
                sl = pl.ds(j * L, L)
                for c in range(C):
                    plsc.store_scatter(out_v, [pfv, vv, zero + c],
                                       ch_v[p, c, sl])
            vb = v0 + t * T
            pltpu.async_copy(out_v.at[p],
                             table_hbm.at[pl.ds(vb, T), :], osem.at[p])
            return carry

        lax.fori_loop(0, NB, block_body, 0)

        for j in range(NB - 2, NB):
            owait(j, j % 2)

    return fmt


def _make_sample_call(N, C, D, H, W):
    PW = N // NW
    NB = PW // B
    NG = B // L

    mesh = plsc.VectorSubcoreMesh(core_axis_name="c", subcore_axis_name="s")

    P = 4
    LEAD = 3

    @functools.partial(
        pl.kernel,
        mesh=mesh,
        compiler_params=_PARAMS,
        out_type=jax.ShapeDtypeStruct((N, C), jnp.float32),
        scratch_types=[
            pltpu.VMEM((6 * L,), jnp.float32),
            pltpu.VMEM((P, B, 3), jnp.float32),
            pltpu.VMEM((P, NC, B), jnp.int32),
            pltpu.VMEM((P, NC, B), jnp.float32),
            pltpu.VMEM((P, NC * B, L), jnp.float32),
            pltpu.VMEM((P, B, C), jnp.float32),
            pltpu.SemaphoreType.DMA((P,)),
            pltpu.SemaphoreType.DMA((P,)),
            pltpu.SemaphoreType.DMA((P,)),
        ],
    )
    def launch(xyz_hbm, table_hbm, params_hbm, out_hbm,
               par_v, xyz_v, idx_v, w_v, rows_v, out_v, gsem, osem, xsem):
        wid = lax.axis_index("s") * 2 + lax.axis_index("c")
        base = wid * PW
        pltpu.sync_copy(params_hbm, par_v)
        iota = lax.iota(jnp.int32, L)
        zero = iota - iota
        mns = [par_v[pl.ds(a * L, L)] for a in range(3)]
        dns = [par_v[pl.ds((3 + a) * L, L)] for a in range(3)]

        def axis_vals(p, a, S):
            n = (p - mns[a]) / dns[a]
            cc = n * 2.0 - 1.0
            u = (cc + 1.0) * 0.5 * float(S - 1)
            u = jnp.clip(u, 0.0, float(S - 1))
            i0 = u.astype(jnp.int32)
            t = u - i0.astype(jnp.float32)
            i1 = jnp.minimum(i0 + 1, S - 1)
            return i0, i1, t

        def xfire(b, p):
            pbase = base + b * B
            pltpu.async_copy(xyz_hbm.at[pl.ds(pbase, B), :], xyz_v.at[p],
                             xsem.at[p])

        def fire(b, p):
            pbase = base + b * B
            pltpu.make_async_copy(xyz_hbm.at[pl.ds(pbase, B), :],
                                  xyz_v.at[p], xsem.at[p]).wait()
            pfv = zero + p
            for g in range(NG):
                off = g * L
                pg = iota + off
                px = plsc.load_gather(xyz_v, [pfv, pg, zero])
                py = plsc.load_gather(xyz_v, [pfv, pg, zero + 1])
                pz = plsc.load_gather(xyz_v, [pfv, pg, zero + 2])
                sl = pl.ds(off, L)
                iz0, iz1, tz = axis_vals(px, 0, D)
                iy0, iy1, ty = axis_vals(py, 1, H)
                ix0, ix1, tx = axis_vals(pz, 2, W)
                a0 = iz0 * (H * W)
                a1 = iz1 * (H * W)
                b0 = iy0 * W
                b1 = iy1 * W
                r00 = a0 + b0
                r01 = a0 + b1
                r10 = a1 + b0
                r11 = a1 + b1
                idx_v[p, 0, sl] = r00 + ix0
                idx_v[p, 1, sl] = r00 + ix1
                idx_v[p, 2, sl] = r01 + ix0
                idx_v[p, 3, sl] = r01 + ix1
                idx_v[p, 4, sl] = r10 + ix0
                idx_v[p, 5, sl] = r10 + ix1
                idx_v[p, 6, sl] = r11 + ix0
                idx_v[p, 7, sl] = r11 + ix1
                cz = 1.0 - tz
                cy = 1.0 - ty
                cx = 1.0 - tx
                w00 = cz * cy
                w01 = cz * ty
                w10 = tz * cy
                w11 = tz * ty
                w_v[p, 0, sl] = w00 * cx
                w_v[p, 1, sl] = w00 * tx
                w_v[p, 2, sl] = w01 * cx
                w_v[p, 3, sl] = w01 * tx
                w_v[p, 4, sl] = w10 * cx
                w_v[p, 5, sl] = w10 * tx
                w_v[p, 6, sl] = w11 * cx
                w_v[p, 7, sl] = w11 * tx
            for k in range(NC):
                pltpu.async_copy(table_hbm.at[idx_v.at[p, k]],
                                 rows_v.at[p, pl.ds(k * B, B), :],
                                 gsem.at[p])

        def gwait(p):
            pltpu.make_async_copy(table_hbm.at[pl.ds(0, NC * B), :],
                                  rows_v.at[p], gsem.at[p]).wait()

        def owait(b, p):
            pbase = base + b * B
            pltpu.make_async_copy(out_v.at[p],
                                  out_hbm.at[pl.ds(pbase, B), :],
                                  osem.at[p]).wait()

        def finish(b, p):
            pbase = base + b * B
            gwait(p)
            pfv = zero + p
            for g in range(NG):
                off = g * L
                pv = iota + off
                sl = pl.ds(off, L)
                wks = [w_v[p, k, sl] for k in range(NC)]
                for ch in range(C):
                    chv = zero + ch
                    gs = [plsc.load_gather(rows_v, [pfv, pv + (k * B), chv])
                          for k in range(NC)]
                    ts = [wks[k] * gs[k] for k in range(NC)]
                    acc = (((ts[0] + ts[1]) + (ts[2] + ts[3])) +
                           ((ts[4] + ts[5]) + (ts[6] + ts[7])))
                    plsc.store_scatter(out_v, [pfv, pv, chv], acc)
            pltpu.async_copy(out_v.at[p],
                             out_hbm.at[pl.ds(pbase, B), :],
                             osem.at[p])

        for j in range(P):
            xfire(j, j)
        for j in range(LEAD):
            fire(j, j)

        def block_body(t, carry):
            p = lax.bitwise_and(t, P - 1)
            pn = lax.bitwise_and(t + LEAD, P - 1)

            @pl.when(t + P < NB)
            def _():
                xfire(t + P, p)

            @pl.when(t + LEAD < NB)
            def _():
                fire(t + LEAD, pn)

            @pl.when(t >= P)
            def _():
                owait(t - P, p)

            finish(t, p)
            return carry

        lax.fori_loop(0, NB, block_body, 0)

        for j in range(NB - P, NB):
            owait(j, j % P)

    return launch


def kernel(xyz, grid, xyz_min, xyz_max):
    N = xyz.shape[0]
    C = grid.shape[1]
    D, H, W = grid.shape[2], grid.shape[3], grid.shape[4]
    V = D * H * W
    grid_flat = grid.reshape(C * V)
    table = _make_format_call(C, V)(grid_flat)
    den = xyz_max - xyz_min
    params = jnp.concatenate([
        jnp.broadcast_to(xyz_min[:, None], (3, L)).reshape(-1),
        jnp.broadcast_to(den[:, None], (3, L)).reshape(-1),
    ])
    call = _make_sample_call(N, C, D, H, W)
    return call(xyz, table, params)
